# baseline (speedup 1.0000x reference)
import jax
import jax.numpy as jnp
from jax.experimental import pallas as pl

_HIDDEN = 256
_NUM_EXPERTS = 8
_TB = 1024


def _moe_block_kernel(x_ref, wg_ref, ws_ref, be_ref, out_ref):
    xb = x_ref[...]
    logits = jnp.dot(xb, wg_ref[...], preferred_element_type=jnp.float32)
    m = jnp.max(logits, axis=-1, keepdims=True)
    ex = jnp.exp(logits - m)
    s = jnp.sum(ex, axis=-1, keepdims=True)
    gate_val = jnp.max(ex, axis=-1, keepdims=True) / s
    idx = jnp.argmax(logits, axis=-1)

    xb16 = xb.astype(jnp.bfloat16)
    zero = jnp.zeros_like(xb16)
    xcat = jnp.concatenate(
        [jnp.where((idx == e)[:, None], xb16, zero) for e in range(_NUM_EXPERTS)],
        axis=1,
    )
    acc = jnp.dot(xcat, ws_ref[...], preferred_element_type=jnp.float32)

    oh = (idx[:, None] == jax.lax.broadcasted_iota(jnp.int32, (_TB, _NUM_EXPERTS), 1))
    bias = jnp.dot(oh.astype(jnp.float32), be_ref[...],
                   preferred_element_type=jnp.float32)
    out_ref[...] = gate_val * (acc + bias)


@jax.jit
def kernel(x, Wg, We, be):
    B, S, H = x.shape
    T = B * S
    xt = x.reshape(T, H)
    ws = We.reshape(_NUM_EXPERTS * H, H).astype(jnp.bfloat16)
    grid = (T // _TB,)
    out = pl.pallas_call(
        _moe_block_kernel,
        grid=grid,
        in_specs=[
            pl.BlockSpec((_TB, H), lambda i: (i, 0)),
            pl.BlockSpec((H, _NUM_EXPERTS), lambda i: (0, 0)),
            pl.BlockSpec((_NUM_EXPERTS * H, H), lambda i: (0, 0)),
            pl.BlockSpec((_NUM_EXPERTS, H), lambda i: (0, 0)),
        ],
        out_specs=pl.BlockSpec((_TB, H), lambda i: (i, 0)),
        out_shape=jax.ShapeDtypeStruct((T, H), jnp.float32),
    )(xt, Wg, ws, be)
    return out.reshape(B, S, H)


# speedup vs baseline: 1.0011x; 1.0011x over previous
import jax
import jax.numpy as jnp
from jax.experimental import pallas as pl

_HIDDEN = 256
_NUM_EXPERTS = 8
_TB = 1024


def _moe_block_kernel(x_ref, wg_ref, ws_ref, be_ref, out_ref):
    xb = x_ref[...]
    logits = jnp.dot(xb, wg_ref[...], preferred_element_type=jnp.float32)
    m = jnp.max(logits, axis=-1, keepdims=True)
    ex = jnp.exp(logits - m)
    s = jnp.sum(ex, axis=-1, keepdims=True)
    gate_val = jnp.max(ex, axis=-1, keepdims=True) / s
    idx = jnp.argmax(logits, axis=-1)

    xb16 = xb.astype(jnp.bfloat16)
    zero = jnp.zeros_like(xb16)
    acc = jnp.zeros((_TB, _HIDDEN), dtype=jnp.float32)
    for e in range(_NUM_EXPERTS):
        xm = jnp.where((idx == e)[:, None], xb16, zero)
        acc = acc + jnp.dot(xm, ws_ref[e], preferred_element_type=jnp.float32)

    oh = (idx[:, None] == jax.lax.broadcasted_iota(jnp.int32, (_TB, _NUM_EXPERTS), 1))
    bias = jnp.dot(oh.astype(jnp.float32), be_ref[...],
                   preferred_element_type=jnp.float32)
    out_ref[...] = gate_val * (acc + bias)


@jax.jit
def kernel(x, Wg, We, be):
    B, S, H = x.shape
    T = B * S
    xt = x.reshape(T, H)
    ws = We.astype(jnp.bfloat16)
    grid = (T // _TB,)
    out = pl.pallas_call(
        _moe_block_kernel,
        grid=grid,
        in_specs=[
            pl.BlockSpec((_TB, H), lambda i: (i, 0)),
            pl.BlockSpec((H, _NUM_EXPERTS), lambda i: (0, 0)),
            pl.BlockSpec((_NUM_EXPERTS, H, H), lambda i: (0, 0, 0)),
            pl.BlockSpec((_NUM_EXPERTS, H), lambda i: (0, 0)),
        ],
        out_specs=pl.BlockSpec((_TB, H), lambda i: (i, 0)),
        out_shape=jax.ShapeDtypeStruct((T, H), jnp.float32),
    )(xt, Wg, ws, be)
    return out.reshape(B, S, H)
